# baseline (speedup 1.0000x reference)
import functools

import jax
import jax.numpy as jnp
from jax import lax
from jax.experimental import pallas as pl
from jax.experimental.pallas import tpu as pltpu
from jax.experimental.pallas import tpu_sc as plsc

D = 256
BATCH = 128
CLIPS = 4
BS = BATCH * CLIPS
NCOL = 1 + (BATCH - 1) * CLIPS
OUTPUT_SIZE = 100000
T = 0.07
MEM_M = 0.5

_HI = lax.Precision.HIGHEST


_NCOPY = 10
_CROWS = OUTPUT_SIZE // _NCOPY


def _tc_body(x_ref, idx_ref, mem_ref, outs_ref, np_ref, cm2_ref, bank_ref,
             sem):
    f32 = jnp.float32
    for cidx in range(_NCOPY):
        sl = pl.ds(cidx * _CROWS, _CROWS)
        pltpu.make_async_copy(mem_ref.at[sl], bank_ref.at[sl], sem).start()

    x = x_ref[...]
    pr = lax.broadcasted_iota(jnp.int32, (BS, BS), 0)
    pc = lax.broadcasted_iota(jnp.int32, (BS, BS), 1)
    perm = (pc == (pr % CLIPS) * BATCH + pr // CLIPS).astype(f32)
    xp = lax.dot_general(perm, x, (((1,), (0,)), ((), ())), precision=_HI)

    g = x[0:128] + x[128:256] + x[256:384] + x[384:512]
    gt = jnp.concatenate([g, g, g, g], axis=0)
    pos_sum = jnp.sum(x * gt, axis=1, keepdims=True)
    selfdot = jnp.sum(x * x, axis=1, keepdims=True)
    pos_mean = (pos_sum - selfdot) * (1.0 / 3.0)
    pos_e = jnp.exp(pos_mean * (1.0 / T))

    sp = lax.dot_general(x, xp, (((1,), (1,)), ((), ())),
                         preferred_element_type=f32)
    e = jnp.exp(sp * (1.0 / T))

    u = lax.broadcasted_iota(jnp.int32, (BS, BS), 1)
    fv = 4 * (lax.broadcasted_iota(jnp.int32, (BS, BS), 0) % BATCH)
    r1 = jnp.concatenate([e[:, BS - 1:BS], e[:, :BS - 1]], axis=1)
    m3 = jnp.concatenate([e[:, 3:], e[:, :3]], axis=1)
    p = jnp.where(u == 0, pos_e, jnp.where(u - 1 < fv, r1, m3))
    p = jnp.where(u < NCOL, p, 0.0)
    z = jnp.sum(p) * (1.0 / (BS * NCOL)) * OUTPUT_SIZE
    outs = p / z
    outs_ref[...] = outs[:, :NCOL]
    rs = jnp.sum(outs, axis=1, keepdims=True)
    np_ref[...] = jnp.mean(outs[:, 0:1] / rs).reshape(1, 1)

    ii = idx_ref[...].astype(f32)
    ri = lax.broadcasted_iota(jnp.int32, (BATCH, BATCH), 0)
    ci = lax.broadcasted_iota(jnp.int32, (BATCH, BATCH), 1)
    eye = (ri == ci).astype(f32)
    idx_col = lax.dot_general(eye, ii, (((1,), (1,)), ((), ())),
                              precision=_HI)
    eqm = idx_col == jnp.broadcast_to(ii, (BATCH, BATCH))
    j = ci.astype(f32)
    last = jnp.max(jnp.where(eqm, j, -1.0), axis=1, keepdims=True)
    oh = (j == last).astype(f32)
    cm2 = lax.dot_general(oh, g, (((1,), (0,)), ((), ())),
                          precision=_HI) * 0.125
    cm2_ref[...] = cm2

    for cidx in range(_NCOPY):
        sl = pl.ds(cidx * _CROWS, _CROWS)
        pltpu.make_async_copy(mem_ref.at[sl], bank_ref.at[sl], sem).wait()


_NW = 16
_BPW = BATCH // _NW


def _sc_body(mem_in, idx_hbm, cm2_hbm, bank, idx_v, rows_v, cm_v, sem):
    wid = lax.axis_index("s")

    pltpu.sync_copy(idx_hbm.at[pl.ds(wid * _BPW, _BPW)], idx_v)
    pltpu.sync_copy(cm2_hbm.at[pl.ds(wid * _BPW, _BPW)], cm_v)
    pltpu.async_copy(mem_in.at[idx_v], rows_v, sem).wait()
    for r in range(_BPW):
        ss = jnp.zeros((16,), jnp.float32)
        for k in range(D // 16):
            sl = pl.ds(k * 16, 16)
            v = rows_v[r, sl] * MEM_M + cm_v[r, sl]
            rows_v[r, sl] = v
            ss = ss + v * v
        tot = jnp.broadcast_to(jnp.sum(ss), (16,))
        ih = plsc.bitcast(tot, jnp.int32)
        y = plsc.bitcast(jnp.int32(0x5F3759DF) - (ih >> 1), jnp.float32)
        for _ in range(4):
            y = y * (1.5 - 0.5 * tot * y * y)
        for k in range(D // 16):
            sl = pl.ds(k * 16, 16)
            rows_v[r, sl] = rows_v[r, sl] * y
    pltpu.async_copy(rows_v, bank.at[idx_v], sem).wait()


@functools.cache
def _sc_update():
    return pl.kernel(
        _sc_body,
        mesh=plsc.VectorSubcoreMesh(core_axis_name="c", subcore_axis_name="s",
                                    num_cores=1),
        compiler_params=pltpu.CompilerParams(needs_layout_passes=False),
        scratch_types=[
            pltpu.VMEM((_BPW,), jnp.int32),
            pltpu.VMEM((_BPW, D), jnp.float32),
            pltpu.VMEM((_BPW, D), jnp.float32),
            pltpu.SemaphoreType.DMA,
        ],
    )


@jax.jit
def kernel(x, memory, idxs, i):
    del i
    idx_tc = idxs.reshape(1, BATCH)
    hbm = pl.BlockSpec(memory_space=pltpu.MemorySpace.HBM)
    vmem = pl.BlockSpec(memory_space=pltpu.MemorySpace.VMEM)
    outs, np11, cm2, bankinit = pl.pallas_call(
        _tc_body,
        in_specs=[vmem, vmem, hbm],
        out_specs=[vmem, vmem, vmem, hbm],
        out_shape=[
            jax.ShapeDtypeStruct((BS, NCOL), jnp.float32),
            jax.ShapeDtypeStruct((1, 1), jnp.float32),
            jax.ShapeDtypeStruct((BATCH, D), jnp.float32),
            jax.ShapeDtypeStruct((OUTPUT_SIZE, D), jnp.float32),
        ],
        scratch_shapes=[pltpu.SemaphoreType.DMA],
    )(x, idx_tc, memory)
    normed_probs = np11.reshape(())

    bank = jax.new_ref(bankinit)
    _sc_update()(memory, idxs, cm2, bank)
    new_memory = jax.freeze(bank)
    return outs, normed_probs, new_memory


# speedup vs baseline: 36.3253x; 36.3253x over previous
import functools

import jax
import jax.numpy as jnp
from jax import lax
from jax.experimental import pallas as pl
from jax.experimental.pallas import tpu as pltpu
from jax.experimental.pallas import tpu_sc as plsc

D = 256
BATCH = 128
CLIPS = 4
BS = BATCH * CLIPS
NCOL = 1 + (BATCH - 1) * CLIPS
OUTPUT_SIZE = 100000
T = 0.07
MEM_M = 0.5

_HI = lax.Precision.HIGHEST


def _tc_body(x_ref, idx_ref, outs_ref, np_ref, cm2_ref):
    f32 = jnp.float32
    x = x_ref[...]
    pr = lax.broadcasted_iota(jnp.int32, (BS, BS), 0)
    pc = lax.broadcasted_iota(jnp.int32, (BS, BS), 1)
    perm = (pc == (pr % CLIPS) * BATCH + pr // CLIPS).astype(f32)
    xp = lax.dot_general(perm, x, (((1,), (0,)), ((), ())), precision=_HI)

    g = x[0:128] + x[128:256] + x[256:384] + x[384:512]
    gt = jnp.concatenate([g, g, g, g], axis=0)
    pos_sum = jnp.sum(x * gt, axis=1, keepdims=True)
    selfdot = jnp.sum(x * x, axis=1, keepdims=True)
    pos_mean = (pos_sum - selfdot) * (1.0 / 3.0)
    pos_e = jnp.exp(pos_mean * (1.0 / T))

    sp = lax.dot_general(x, xp, (((1,), (1,)), ((), ())),
                         preferred_element_type=f32)
    e = jnp.exp(sp * (1.0 / T))

    u = lax.broadcasted_iota(jnp.int32, (BS, BS), 1)
    fv = 4 * (lax.broadcasted_iota(jnp.int32, (BS, BS), 0) % BATCH)
    r1 = jnp.concatenate([e[:, BS - 1:BS], e[:, :BS - 1]], axis=1)
    m3 = jnp.concatenate([e[:, 3:], e[:, :3]], axis=1)
    p = jnp.where(u == 0, pos_e, jnp.where(u - 1 < fv, r1, m3))
    p = jnp.where(u < NCOL, p, 0.0)
    z = jnp.sum(p) * (1.0 / (BS * NCOL)) * OUTPUT_SIZE
    outs = p / z
    outs_ref[...] = outs[:, :NCOL]
    rs = jnp.sum(outs, axis=1, keepdims=True)
    np_ref[...] = jnp.mean(outs[:, 0:1] / rs).reshape(1, 1)

    ii = idx_ref[...].astype(f32)
    ri = lax.broadcasted_iota(jnp.int32, (BATCH, BATCH), 0)
    ci = lax.broadcasted_iota(jnp.int32, (BATCH, BATCH), 1)
    eye = (ri == ci).astype(f32)
    idx_col = lax.dot_general(eye, ii, (((1,), (1,)), ((), ())),
                              precision=_HI)
    eqm = idx_col == jnp.broadcast_to(ii, (BATCH, BATCH))
    j = ci.astype(f32)
    last = jnp.max(jnp.where(eqm, j, -1.0), axis=1, keepdims=True)
    oh = (j == last).astype(f32)
    cm2 = lax.dot_general(oh, g, (((1,), (0,)), ((), ())),
                          precision=_HI) * 0.125
    cm2_ref[...] = cm2


_NW = 16
_BPW = BATCH // _NW


def _sc_body(mem_in, idx_hbm, cm2_hbm, bank, idx_v, rows_v, cm_v, sem):
    wid = lax.axis_index("s")

    pltpu.sync_copy(idx_hbm.at[pl.ds(wid * _BPW, _BPW)], idx_v)
    pltpu.sync_copy(cm2_hbm.at[pl.ds(wid * _BPW, _BPW)], cm_v)
    pltpu.async_copy(mem_in.at[idx_v], rows_v, sem).wait()
    for r in range(_BPW):
        ss = jnp.zeros((16,), jnp.float32)
        for k in range(D // 16):
            sl = pl.ds(k * 16, 16)
            v = rows_v[r, sl] * MEM_M + cm_v[r, sl]
            rows_v[r, sl] = v
            ss = ss + v * v
        tot = jnp.broadcast_to(jnp.sum(ss), (16,))
        ih = plsc.bitcast(tot, jnp.int32)
        y = plsc.bitcast(jnp.int32(0x5F3759DF) - (ih >> 1), jnp.float32)
        for _ in range(4):
            y = y * (1.5 - 0.5 * tot * y * y)
        for k in range(D // 16):
            sl = pl.ds(k * 16, 16)
            rows_v[r, sl] = rows_v[r, sl] * y
    pltpu.async_copy(rows_v, bank.at[idx_v], sem).wait()


@functools.cache
def _sc_update():
    return pl.kernel(
        _sc_body,
        mesh=plsc.VectorSubcoreMesh(core_axis_name="c", subcore_axis_name="s",
                                    num_cores=1),
        compiler_params=pltpu.CompilerParams(needs_layout_passes=False),
        scratch_types=[
            pltpu.VMEM((_BPW,), jnp.int32),
            pltpu.VMEM((_BPW, D), jnp.float32),
            pltpu.VMEM((_BPW, D), jnp.float32),
            pltpu.SemaphoreType.DMA,
        ],
    )


@jax.jit
def kernel(x, memory, idxs, i):
    del i
    idx_tc = idxs.reshape(1, BATCH)
    outs, np11, cm2 = pl.pallas_call(
        _tc_body,
        out_shape=[
            jax.ShapeDtypeStruct((BS, NCOL), jnp.float32),
            jax.ShapeDtypeStruct((1, 1), jnp.float32),
            jax.ShapeDtypeStruct((BATCH, D), jnp.float32),
        ],
    )(x, idx_tc)
    normed_probs = np11.reshape(())

    bank = jax.new_ref(memory)
    _sc_update()(memory, idxs, cm2, bank)
    new_memory = jax.freeze(bank)
    return outs, normed_probs, new_memory


# speedup vs baseline: 36.4230x; 1.0027x over previous
import functools

import jax
import jax.numpy as jnp
from jax import lax
from jax.experimental import pallas as pl
from jax.experimental.pallas import tpu as pltpu
from jax.experimental.pallas import tpu_sc as plsc

D = 256
BATCH = 128
CLIPS = 4
BS = BATCH * CLIPS
NCOL = 1 + (BATCH - 1) * CLIPS
OUTPUT_SIZE = 100000
T = 0.07
MEM_M = 0.5

_HI = lax.Precision.HIGHEST


def _tc_body(x_ref, idx_ref, outs_ref, np_ref, cm2_ref):
    f32 = jnp.float32
    x = x_ref[...]
    pr = lax.broadcasted_iota(jnp.int32, (BS, BS), 0)
    pc = lax.broadcasted_iota(jnp.int32, (BS, BS), 1)
    perm = (pc == (pr % CLIPS) * BATCH + pr // CLIPS).astype(f32)
    xp = lax.dot_general(perm, x, (((1,), (0,)), ((), ())),
                         preferred_element_type=f32)

    g = x[0:128] + x[128:256] + x[256:384] + x[384:512]
    gt = jnp.concatenate([g, g, g, g], axis=0)
    pos_sum = jnp.sum(x * gt, axis=1, keepdims=True)
    selfdot = jnp.sum(x * x, axis=1, keepdims=True)
    pos_mean = (pos_sum - selfdot) * (1.0 / 3.0)
    pos_e = jnp.exp(pos_mean * (1.0 / T))

    sp = lax.dot_general(x, xp, (((1,), (1,)), ((), ())),
                         preferred_element_type=f32)
    e = jnp.exp(sp * (1.0 / T))

    u = lax.broadcasted_iota(jnp.int32, (BS, BS), 1)
    fv = 4 * (lax.broadcasted_iota(jnp.int32, (BS, BS), 0) % BATCH)
    r1 = jnp.concatenate([e[:, BS - 1:BS], e[:, :BS - 1]], axis=1)
    m3 = jnp.concatenate([e[:, 3:], e[:, :3]], axis=1)
    p = jnp.where(u == 0, pos_e, jnp.where(u - 1 < fv, r1, m3))
    p = jnp.where(u < NCOL, p, 0.0)
    z = jnp.sum(p) * (1.0 / (BS * NCOL)) * OUTPUT_SIZE
    outs = p / z
    outs_ref[...] = outs[:, :NCOL]
    rs = jnp.sum(outs, axis=1, keepdims=True)
    np_ref[...] = jnp.mean(outs[:, 0:1] / rs).reshape(1, 1)

    ii = idx_ref[...].astype(f32)
    ri = lax.broadcasted_iota(jnp.int32, (BATCH, BATCH), 0)
    ci = lax.broadcasted_iota(jnp.int32, (BATCH, BATCH), 1)
    eye = (ri == ci).astype(f32)
    idx_col = lax.dot_general(eye, ii, (((1,), (1,)), ((), ())),
                              precision=_HI)
    eqm = idx_col == jnp.broadcast_to(ii, (BATCH, BATCH))
    j = ci.astype(f32)
    last = jnp.max(jnp.where(eqm, j, -1.0), axis=1, keepdims=True)
    oh = (j == last).astype(f32)
    cm2 = lax.dot_general(oh, g, (((1,), (0,)), ((), ())),
                          precision=_HI) * 0.125
    cm2_ref[...] = cm2


_NW = 16
_BPW = BATCH // _NW


def _sc_body(mem_in, idx_hbm, cm2_hbm, bank, idx_v, rows_v, cm_v, sem):
    wid = lax.axis_index("s")

    pltpu.sync_copy(idx_hbm.at[pl.ds(wid * _BPW, _BPW)], idx_v)
    pltpu.sync_copy(cm2_hbm.at[pl.ds(wid * _BPW, _BPW)], cm_v)
    pltpu.async_copy(mem_in.at[idx_v], rows_v, sem).wait()
    for r in range(_BPW):
        ss = jnp.zeros((16,), jnp.float32)
        for k in range(D // 16):
            sl = pl.ds(k * 16, 16)
            v = rows_v[r, sl] * MEM_M + cm_v[r, sl]
            rows_v[r, sl] = v
            ss = ss + v * v
        tot = jnp.broadcast_to(jnp.sum(ss), (16,))
        ih = plsc.bitcast(tot, jnp.int32)
        y = plsc.bitcast(jnp.int32(0x5F3759DF) - (ih >> 1), jnp.float32)
        for _ in range(4):
            y = y * (1.5 - 0.5 * tot * y * y)
        for k in range(D // 16):
            sl = pl.ds(k * 16, 16)
            rows_v[r, sl] = rows_v[r, sl] * y
    pltpu.async_copy(rows_v, bank.at[idx_v], sem).wait()


@functools.cache
def _sc_update():
    return pl.kernel(
        _sc_body,
        mesh=plsc.VectorSubcoreMesh(core_axis_name="c", subcore_axis_name="s",
                                    num_cores=1),
        compiler_params=pltpu.CompilerParams(needs_layout_passes=False),
        scratch_types=[
            pltpu.VMEM((_BPW,), jnp.int32),
            pltpu.VMEM((_BPW, D), jnp.float32),
            pltpu.VMEM((_BPW, D), jnp.float32),
            pltpu.SemaphoreType.DMA,
        ],
    )


@jax.jit
def kernel(x, memory, idxs, i):
    del i
    idx_tc = idxs.reshape(1, BATCH)
    outs, np11, cm2 = pl.pallas_call(
        _tc_body,
        out_shape=[
            jax.ShapeDtypeStruct((BS, NCOL), jnp.float32),
            jax.ShapeDtypeStruct((1, 1), jnp.float32),
            jax.ShapeDtypeStruct((BATCH, D), jnp.float32),
        ],
    )(x, idx_tc)
    normed_probs = np11.reshape(())

    bank = jax.new_ref(memory)
    _sc_update()(memory, idxs, cm2, bank)
    new_memory = jax.freeze(bank)
    return outs, normed_probs, new_memory


# speedup vs baseline: 36.7908x; 1.0101x over previous
import functools

import jax
import jax.numpy as jnp
from jax import lax
from jax.experimental import pallas as pl
from jax.experimental.pallas import tpu as pltpu
from jax.experimental.pallas import tpu_sc as plsc

D = 256
BATCH = 128
CLIPS = 4
BS = BATCH * CLIPS
NCOL = 1 + (BATCH - 1) * CLIPS
OUTPUT_SIZE = 100000
T = 0.07
MEM_M = 0.5

_HI = lax.Precision.HIGHEST


def _tc_body(x_ref, idx_ref, outs_ref, np_ref, cm2_ref):
    f32 = jnp.float32
    x = x_ref[...]
    pr = lax.broadcasted_iota(jnp.int32, (BS, BS), 0)
    pc = lax.broadcasted_iota(jnp.int32, (BS, BS), 1)
    perm = (pc == (pr % CLIPS) * BATCH + pr // CLIPS).astype(f32)
    xp = lax.dot_general(perm, x, (((1,), (0,)), ((), ())),
                         preferred_element_type=f32)

    g = x[0:128] + x[128:256] + x[256:384] + x[384:512]
    gt = jnp.concatenate([g, g, g, g], axis=0)
    pos_sum = jnp.sum(x * gt, axis=1, keepdims=True)
    selfdot = jnp.sum(x * x, axis=1, keepdims=True)
    pos_mean = (pos_sum - selfdot) * (1.0 / 3.0)
    pos_e = jnp.exp(pos_mean * (1.0 / T))

    sp = lax.dot_general(x, xp, (((1,), (1,)), ((), ())),
                         preferred_element_type=f32)
    e = jnp.exp(sp * (1.0 / T))

    u = lax.broadcasted_iota(jnp.int32, (BS, BS), 1)
    fv = 4 * (lax.broadcasted_iota(jnp.int32, (BS, BS), 0) % BATCH)
    r1 = jnp.concatenate([e[:, BS - 1:BS], e[:, :BS - 1]], axis=1)
    m3 = jnp.concatenate([e[:, 3:], e[:, :3]], axis=1)
    p = jnp.where(u == 0, pos_e, jnp.where(u - 1 < fv, r1, m3))
    p = jnp.where(u < NCOL, p, 0.0)
    z = jnp.sum(p) * (1.0 / (BS * NCOL)) * OUTPUT_SIZE
    outs = p / z
    outs_ref[...] = outs[:, :NCOL]
    rs = jnp.sum(outs, axis=1, keepdims=True)
    np_ref[...] = jnp.mean(outs[:, 0:1] / rs).reshape(1, 1)

    ii = idx_ref[...].astype(f32)
    ri = lax.broadcasted_iota(jnp.int32, (BATCH, BATCH), 0)
    ci = lax.broadcasted_iota(jnp.int32, (BATCH, BATCH), 1)
    eye = (ri == ci).astype(f32)
    idx_col = lax.dot_general(eye, ii, (((1,), (1,)), ((), ())),
                              precision=_HI)
    eqm = idx_col == jnp.broadcast_to(ii, (BATCH, BATCH))
    j = ci.astype(f32)
    last = jnp.max(jnp.where(eqm, j, -1.0), axis=1, keepdims=True)
    oh = (j == last).astype(f32)
    cm2 = lax.dot_general(oh, g, (((1,), (0,)), ((), ())),
                          precision=_HI) * 0.125
    cm2_ref[...] = cm2


_NW = 16
_BPW = BATCH // _NW


def _sc_body(mem_in, idx_hbm, cm2_hbm, bank, idx_v, rows_v, cm_v, sem, sem2):
    wid = lax.axis_index("s")

    pltpu.sync_copy(idx_hbm.at[pl.ds(wid * _BPW, _BPW)], idx_v)
    cm_cp = pltpu.async_copy(cm2_hbm.at[pl.ds(wid * _BPW, _BPW)], cm_v, sem2)
    pltpu.async_copy(mem_in.at[idx_v], rows_v, sem).wait()
    cm_cp.wait()
    for r in range(_BPW):
        ss = jnp.zeros((16,), jnp.float32)
        for k in range(D // 16):
            sl = pl.ds(k * 16, 16)
            v = rows_v[r, sl] * MEM_M + cm_v[r, sl]
            rows_v[r, sl] = v
            ss = ss + v * v
        tot = jnp.broadcast_to(jnp.sum(ss), (16,))
        ih = plsc.bitcast(tot, jnp.int32)
        y = plsc.bitcast(jnp.int32(0x5F3759DF) - (ih >> 1), jnp.float32)
        for _ in range(4):
            y = y * (1.5 - 0.5 * tot * y * y)
        for k in range(D // 16):
            sl = pl.ds(k * 16, 16)
            rows_v[r, sl] = rows_v[r, sl] * y
    pltpu.async_copy(rows_v, bank.at[idx_v], sem).wait()


@functools.cache
def _sc_update():
    return pl.kernel(
        _sc_body,
        mesh=plsc.VectorSubcoreMesh(core_axis_name="c", subcore_axis_name="s",
                                    num_cores=1),
        compiler_params=pltpu.CompilerParams(needs_layout_passes=False),
        scratch_types=[
            pltpu.VMEM((_BPW,), jnp.int32),
            pltpu.VMEM((_BPW, D), jnp.float32),
            pltpu.VMEM((_BPW, D), jnp.float32),
            pltpu.SemaphoreType.DMA,
            pltpu.SemaphoreType.DMA,
        ],
    )


@jax.jit
def kernel(x, memory, idxs, i):
    del i
    idx_tc = idxs.reshape(1, BATCH)
    outs, np11, cm2 = pl.pallas_call(
        _tc_body,
        out_shape=[
            jax.ShapeDtypeStruct((BS, NCOL), jnp.float32),
            jax.ShapeDtypeStruct((1, 1), jnp.float32),
            jax.ShapeDtypeStruct((BATCH, D), jnp.float32),
        ],
    )(x, idx_tc)
    normed_probs = np11.reshape(())

    bank = jax.new_ref(memory)
    _sc_update()(memory, idxs, cm2, bank)
    new_memory = jax.freeze(bank)
    return outs, normed_probs, new_memory
